# baseline (speedup 1.0000x reference)
import functools

import jax
import jax.numpy as jnp
from jax import lax
from jax.experimental import pallas as pl
from jax.experimental.pallas import tpu as pltpu
from jax.experimental.pallas import tpu_sc as plsc

BATCH = 1024
SENT_LEN = 200
WORD_DIM = 64
WORD_PAD = 128
FEAT_DIM = 32
LABEL_VOCAB = 50
LABEL_DIM = 128
OUT_DIM = WORD_DIM + 2 * FEAT_DIM

NUM_WORKERS = 32

TOKENS = BATCH * SENT_LEN
TOK_PER_W = TOKENS // NUM_WORKERS
TOK_CHUNK = 128
TOK_NCHUNK = TOK_PER_W // TOK_CHUNK

LABELS = BATCH * LABEL_VOCAB
LAB_PER_W = LABELS // NUM_WORKERS
LAB_CHUNK = 32
LAB_NCHUNK = LAB_PER_W // LAB_CHUNK

NBUF = 5
LOOKAHEAD = 2

_mesh = plsc.VectorSubcoreMesh(core_axis_name="c", subcore_axis_name="s")


def _ring_pipeline(nchunk, gather, wait_gather, write, wait_write):
    for b in range(LOOKAHEAD):
        gather(b, b)

    def body(c0, carry):
        for b in range(NBUF):
            c = c0 + b
            wait_gather(b)
            write(c, b)
            g = c + LOOKAHEAD
            gb = (b + LOOKAHEAD) % NBUF

            @pl.when(g < nchunk)
            def _():
                @pl.when(g >= NBUF)
                def _():
                    wait_write(gb)
                gather(g, gb)
        return carry

    lax.fori_loop(0, nchunk // NBUF, lambda i, cr: body(i * NBUF, cr), 0)

    for k in range(nchunk - NBUF, nchunk):
        wait_write(k % NBUF)


@functools.partial(
    pl.kernel,
    out_type=jax.ShapeDtypeStruct((8, WORD_DIM), jnp.float32),
    mesh=_mesh,
    scratch_types=[pltpu.VMEM((8, WORD_DIM), jnp.float32)],
)
def _touch_sc(wtab_hbm, out8, buf):
    pltpu.sync_copy(wtab_hbm.at[pl.ds(0, 8)], buf)
    pltpu.sync_copy(buf, out8)


@functools.partial(
    pl.kernel,
    out_type=jax.ShapeDtypeStruct((LABELS, LABEL_DIM), jnp.float32),
    mesh=_mesh,
    compiler_params=pltpu.CompilerParams(use_tc_tiling_on_sc=False),
    scratch_types=[
        pltpu.VMEM((TOK_PER_W,), jnp.int32),
        pltpu.VMEM((TOK_PER_W,), jnp.int32),
        pltpu.VMEM((LAB_PER_W,), jnp.int32),
        pltpu.VMEM((8, WORD_DIM), jnp.float32),
        pltpu.VMEM((NBUF, TOK_CHUNK, FEAT_DIM), jnp.float32),
        pltpu.VMEM((NBUF, TOK_CHUNK, FEAT_DIM), jnp.float32),
        pltpu.VMEM((NBUF, LAB_CHUNK, LABEL_DIM), jnp.float32),
        [pltpu.SemaphoreType.DMA] * NBUF,
        [pltpu.SemaphoreType.DMA] * NBUF,
        [pltpu.SemaphoreType.DMA] * NBUF,
        [pltpu.SemaphoreType.DMA] * NBUF,
    ],
)
def _feat_label_sc(f0idx_hbm, f1idx_hbm, lidx_hbm,
                   f0tab_hbm, f1tab_hbm, ltab_hbm, dep8_hbm, oref,
                   out_label,
                   f0idx_v, f1idx_v, lidx_v, dep_v,
                   rows_f0, rows_f1, lrows, sem_g, sem_w, lsem_g, lsem_w):
    wid = lax.axis_index("s") * 2 + lax.axis_index("c")

    pltpu.sync_copy(dep8_hbm, dep_v)

    pltpu.sync_copy(f0idx_hbm.at[pl.ds(wid * TOK_PER_W, TOK_PER_W)], f0idx_v)
    pltpu.sync_copy(f1idx_hbm.at[pl.ds(wid * TOK_PER_W, TOK_PER_W)], f1idx_v)
    pltpu.sync_copy(lidx_hbm.at[pl.ds(wid * LAB_PER_W, LAB_PER_W)], lidx_v)

    tok_base = wid * TOK_PER_W
    lab_base = wid * LAB_PER_W

    def tok_gather(c, b):
        sl = pl.ds(c * TOK_CHUNK, TOK_CHUNK)
        pltpu.async_copy(f0tab_hbm.at[f0idx_v.at[sl]], rows_f0.at[b],
                         sem_g[b])
        pltpu.async_copy(f1tab_hbm.at[f1idx_v.at[sl]], rows_f1.at[b],
                         sem_g[b])

    def tok_wait_gather(b):
        pltpu.make_async_copy(oref.at[pl.ds(0, TOK_CHUNK)],
                              rows_f0.at[b], sem_g[b]).wait()
        pltpu.make_async_copy(oref.at[pl.ds(0, TOK_CHUNK)],
                              rows_f1.at[b], sem_g[b]).wait()

    def tok_write(c, b):
        dst = tok_base + c * TOK_CHUNK
        pltpu.async_copy(rows_f0.at[b],
                         oref.at[pl.ds(dst, TOK_CHUNK),
                                 pl.ds(WORD_DIM, FEAT_DIM)], sem_w[b])
        pltpu.async_copy(rows_f1.at[b],
                         oref.at[pl.ds(dst, TOK_CHUNK),
                                 pl.ds(WORD_DIM + FEAT_DIM, FEAT_DIM)],
                         sem_w[b])

    def tok_wait_write(b):
        pltpu.make_async_copy(rows_f0.at[b],
                              oref.at[pl.ds(0, TOK_CHUNK),
                                      pl.ds(WORD_DIM, FEAT_DIM)],
                              sem_w[b]).wait()
        pltpu.make_async_copy(rows_f1.at[b],
                              oref.at[pl.ds(0, TOK_CHUNK),
                                      pl.ds(WORD_DIM + FEAT_DIM, FEAT_DIM)],
                              sem_w[b]).wait()

    def lab_gather(c, b):
        sl = pl.ds(c * LAB_CHUNK, LAB_CHUNK)
        pltpu.async_copy(ltab_hbm.at[lidx_v.at[sl]], lrows.at[b], lsem_g[b])

    def lab_wait_gather(b):
        pltpu.make_async_copy(out_label.at[pl.ds(0, LAB_CHUNK)], lrows.at[b],
                              lsem_g[b]).wait()

    def lab_write(c, b):
        pltpu.async_copy(lrows.at[b],
                         out_label.at[pl.ds(lab_base + c * LAB_CHUNK,
                                            LAB_CHUNK)], lsem_w[b])

    def lab_wait_write(b):
        pltpu.make_async_copy(lrows.at[b], out_label.at[pl.ds(0, LAB_CHUNK)],
                              lsem_w[b]).wait()

    _ring_pipeline(TOK_NCHUNK, tok_gather, tok_wait_gather, tok_write,
                   tok_wait_write)
    _ring_pipeline(LAB_NCHUNK, lab_gather, lab_wait_gather, lab_write,
                   lab_wait_write)


@functools.partial(
    pl.kernel,
    out_type=(),
    mesh=_mesh,
    compiler_params=pltpu.CompilerParams(use_tc_tiling_on_sc=False),
    scratch_types=[
        pltpu.VMEM((TOK_PER_W,), jnp.int32),
        pltpu.VMEM((NBUF, TOK_CHUNK, WORD_DIM), jnp.float32),
        [pltpu.SemaphoreType.DMA] * NBUF,
        [pltpu.SemaphoreType.DMA] * NBUF,
    ],
)
def _word_sc(widx_hbm, wtab_hbm, oref,
             widx_v, rows_w, sem_g, sem_w):
    wid = lax.axis_index("s") * 2 + lax.axis_index("c")

    pltpu.sync_copy(widx_hbm.at[pl.ds(wid * TOK_PER_W, TOK_PER_W)], widx_v)

    tok_base = wid * TOK_PER_W

    def gather(c, b):
        sl = pl.ds(c * TOK_CHUNK, TOK_CHUNK)
        pltpu.async_copy(wtab_hbm.at[widx_v.at[sl]], rows_w.at[b], sem_g[b])

    def wait_gather(b):
        pltpu.make_async_copy(oref.at[pl.ds(0, TOK_CHUNK),
                                      pl.ds(0, WORD_DIM)],
                              rows_w.at[b], sem_g[b]).wait()

    def write(c, b):
        pltpu.async_copy(rows_w.at[b],
                         oref.at[pl.ds(tok_base + c * TOK_CHUNK, TOK_CHUNK),
                                 pl.ds(0, WORD_DIM)], sem_w[b])

    def wait_write(b):
        pltpu.make_async_copy(rows_w.at[b],
                              oref.at[pl.ds(0, TOK_CHUNK),
                                      pl.ds(0, WORD_DIM)],
                              sem_w[b]).wait()

    _ring_pipeline(TOK_NCHUNK, gather, wait_gather, write, wait_write)


def kernel(word_inputs, feature_inputs, word_seq_lengths, char_inputs,
           char_seq_lengths, char_seq_recover, input_label_seq_tensor,
           word_table, feat_tables, label_table):
    widx = word_inputs.astype(jnp.int32).reshape(TOKENS)
    f0idx = feature_inputs[0].astype(jnp.int32).reshape(TOKENS)
    f1idx = feature_inputs[1].astype(jnp.int32).reshape(TOKENS)
    lidx = input_label_seq_tensor.astype(jnp.int32).reshape(LABELS)

    o_ref = jax.empty_ref(jax.ShapeDtypeStruct((TOKENS, OUT_DIM),
                                               jnp.float32))
    dep8 = _touch_sc(word_table)
    out_label = _feat_label_sc(f0idx, f1idx, lidx,
                               feat_tables[0], feat_tables[1], label_table,
                               dep8, o_ref)
    _word_sc(widx, word_table, o_ref)
    out_word = o_ref[...]

    word_represent = out_word.reshape(BATCH, SENT_LEN, OUT_DIM)
    label_embs = out_label.reshape(BATCH, LABEL_VOCAB, LABEL_DIM)
    return (word_represent, label_embs)


# speedup vs baseline: 1.1777x; 1.1777x over previous
import functools

import jax
import jax.numpy as jnp
from jax import lax
from jax.experimental import pallas as pl
from jax.experimental.pallas import tpu as pltpu
from jax.experimental.pallas import tpu_sc as plsc

BATCH = 1024
SENT_LEN = 200
WORD_DIM = 64
FEAT_DIM = 32
LABEL_VOCAB = 50
LABEL_DIM = 128
OUT_DIM = WORD_DIM + 2 * FEAT_DIM

NUM_WORKERS = 32

TOKENS = BATCH * SENT_LEN
TOK_PER_W = TOKENS // NUM_WORKERS
TOK_CHUNK = 128
TOK_NCHUNK = TOK_PER_W // TOK_CHUNK

LABELS = BATCH * LABEL_VOCAB
LAB_PER_W = LABELS // NUM_WORKERS
LAB_CHUNK = 32
LAB_NCHUNK = LAB_PER_W // LAB_CHUNK

NBUF = 5
LOOKAHEAD = 3

_mesh = plsc.VectorSubcoreMesh(core_axis_name="c", subcore_axis_name="s")


@functools.partial(
    pl.kernel,
    out_type=(
        jax.ShapeDtypeStruct((TOKENS, OUT_DIM), jnp.float32),
        jax.ShapeDtypeStruct((LABELS, LABEL_DIM), jnp.float32),
    ),
    mesh=_mesh,
    compiler_params=pltpu.CompilerParams(use_tc_tiling_on_sc=False),
    scratch_types=[
        pltpu.VMEM((TOK_PER_W,), jnp.int32),
        pltpu.VMEM((TOK_PER_W,), jnp.int32),
        pltpu.VMEM((TOK_PER_W,), jnp.int32),
        pltpu.VMEM((LAB_PER_W,), jnp.int32),
        pltpu.VMEM((NBUF, TOK_CHUNK, WORD_DIM), jnp.float32),
        pltpu.VMEM((NBUF, TOK_CHUNK, FEAT_DIM), jnp.float32),
        pltpu.VMEM((NBUF, TOK_CHUNK, FEAT_DIM), jnp.float32),
        pltpu.VMEM((NBUF, LAB_CHUNK, LABEL_DIM), jnp.float32),
        [pltpu.SemaphoreType.DMA] * NBUF,
        [pltpu.SemaphoreType.DMA] * NBUF,
        [pltpu.SemaphoreType.DMA] * NBUF,
        [pltpu.SemaphoreType.DMA] * NBUF,
    ],
)
def _wordrep_sc(widx_hbm, f0idx_hbm, f1idx_hbm, lidx_hbm,
                wtab_hbm, f0tab_hbm, f1tab_hbm, ltab_hbm,
                out_word, out_label,
                widx_v, f0idx_v, f1idx_v, lidx_v,
                rows_w, rows_f0, rows_f1, lrows, sem_g, sem_w,
                lsem_g, lsem_w):
    wid = lax.axis_index("s") * 2 + lax.axis_index("c")

    pltpu.sync_copy(widx_hbm.at[pl.ds(wid * TOK_PER_W, TOK_PER_W)], widx_v)
    pltpu.sync_copy(f0idx_hbm.at[pl.ds(wid * TOK_PER_W, TOK_PER_W)], f0idx_v)
    pltpu.sync_copy(f1idx_hbm.at[pl.ds(wid * TOK_PER_W, TOK_PER_W)], f1idx_v)
    pltpu.sync_copy(lidx_hbm.at[pl.ds(wid * LAB_PER_W, LAB_PER_W)], lidx_v)

    tok_base = wid * TOK_PER_W
    lab_base = wid * LAB_PER_W

    def tok_gather(c, b):
        sl = pl.ds(c * TOK_CHUNK, TOK_CHUNK)
        pltpu.async_copy(wtab_hbm.at[widx_v.at[sl]], rows_w.at[b], sem_g[b])
        pltpu.async_copy(f0tab_hbm.at[f0idx_v.at[sl]], rows_f0.at[b],
                         sem_g[b])
        pltpu.async_copy(f1tab_hbm.at[f1idx_v.at[sl]], rows_f1.at[b],
                         sem_g[b])

    def tok_wait_gather(b):
        pltpu.make_async_copy(out_word.at[pl.ds(0, TOK_CHUNK)],
                              rows_w.at[b], sem_g[b]).wait()
        pltpu.make_async_copy(out_word.at[pl.ds(0, TOK_CHUNK)],
                              rows_f0.at[b], sem_g[b]).wait()
        pltpu.make_async_copy(out_word.at[pl.ds(0, TOK_CHUNK)],
                              rows_f1.at[b], sem_g[b]).wait()

    def tok_write(c, b):
        dst = tok_base + c * TOK_CHUNK
        pltpu.async_copy(rows_w.at[b],
                         out_word.at[pl.ds(dst, TOK_CHUNK),
                                     pl.ds(0, WORD_DIM)], sem_w[b])
        pltpu.async_copy(rows_f0.at[b],
                         out_word.at[pl.ds(dst, TOK_CHUNK),
                                     pl.ds(WORD_DIM, FEAT_DIM)], sem_w[b])
        pltpu.async_copy(rows_f1.at[b],
                         out_word.at[pl.ds(dst, TOK_CHUNK),
                                     pl.ds(WORD_DIM + FEAT_DIM, FEAT_DIM)],
                         sem_w[b])

    def tok_wait_write(b):
        pltpu.make_async_copy(rows_w.at[b],
                              out_word.at[pl.ds(0, TOK_CHUNK),
                                          pl.ds(0, WORD_DIM)],
                              sem_w[b]).wait()
        pltpu.make_async_copy(rows_f0.at[b],
                              out_word.at[pl.ds(0, TOK_CHUNK),
                                          pl.ds(WORD_DIM, FEAT_DIM)],
                              sem_w[b]).wait()
        pltpu.make_async_copy(rows_f1.at[b],
                              out_word.at[pl.ds(0, TOK_CHUNK),
                                          pl.ds(WORD_DIM + FEAT_DIM,
                                                FEAT_DIM)],
                              sem_w[b]).wait()

    def lab_gather(c, b):
        sl = pl.ds(c * LAB_CHUNK, LAB_CHUNK)
        pltpu.async_copy(ltab_hbm.at[lidx_v.at[sl]], lrows.at[b], lsem_g[b])

    def lab_wait_gather(b):
        pltpu.make_async_copy(out_label.at[pl.ds(0, LAB_CHUNK)], lrows.at[b],
                              lsem_g[b]).wait()

    def lab_write(c, b):
        pltpu.async_copy(lrows.at[b],
                         out_label.at[pl.ds(lab_base + c * LAB_CHUNK,
                                            LAB_CHUNK)], lsem_w[b])

    def lab_wait_write(b):
        pltpu.make_async_copy(lrows.at[b], out_label.at[pl.ds(0, LAB_CHUNK)],
                              lsem_w[b]).wait()

    def pipeline(nchunk, gather, wait_gather, write, wait_write):
        for b in range(LOOKAHEAD):
            gather(b, b)

        def body(c0, carry):
            for b in range(NBUF):
                c = c0 + b
                wait_gather(b)
                write(c, b)
                g = c + LOOKAHEAD
                gb = (b + LOOKAHEAD) % NBUF

                @pl.when(g < nchunk)
                def _():
                    @pl.when(g >= NBUF)
                    def _():
                        wait_write(gb)
                    gather(g, gb)
            return carry

        lax.fori_loop(0, nchunk // NBUF, lambda i, cr: body(i * NBUF, cr), 0)

        for k in range(nchunk - NBUF, nchunk):
            wait_write(k % NBUF)

    pipeline(TOK_NCHUNK, tok_gather, tok_wait_gather, tok_write,
             tok_wait_write)
    pipeline(LAB_NCHUNK, lab_gather, lab_wait_gather, lab_write,
             lab_wait_write)


def kernel(word_inputs, feature_inputs, word_seq_lengths, char_inputs,
           char_seq_lengths, char_seq_recover, input_label_seq_tensor,
           word_table, feat_tables, label_table):
    widx = word_inputs.astype(jnp.int32).reshape(TOKENS)
    f0idx = feature_inputs[0].astype(jnp.int32).reshape(TOKENS)
    f1idx = feature_inputs[1].astype(jnp.int32).reshape(TOKENS)
    lidx = input_label_seq_tensor.astype(jnp.int32).reshape(LABELS)

    out_word, out_label = _wordrep_sc(
        widx, f0idx, f1idx, lidx,
        word_table, feat_tables[0], feat_tables[1], label_table)

    word_represent = out_word.reshape(BATCH, SENT_LEN, OUT_DIM)
    label_embs = out_label.reshape(BATCH, LABEL_VOCAB, LABEL_DIM)
    return (word_represent, label_embs)
